# baseline (speedup 1.0000x reference)
import functools

import jax
import jax.numpy as jnp
import numpy as np
from jax import lax
from jax.experimental import pallas as pl
from jax.experimental.pallas import tpu as pltpu
from jax.experimental.pallas import tpu_sc as plsc

N = 10000
E = 320000
NCRY = 512
NCONV = 3

NCORE = 2
NSUB = 16
NW = NCORE * NSUB

E_PAD = 327680
EPW = E_PAD // NW
GC = 128

N_TAB = 10240
NROWS_SUB = N_TAB // NSUB

CRY_TAB = 3 * NCRY
CRY_ROWS_SUB = CRY_TAB // NSUB
X_ROWS = 10240
XPW = X_ROWS // NW
XC = 64

EBLK = 1280
E_HALF = E_PAD // 2
EPW_H = E_HALF // NW
N_EBLK_H = E_HALF // EBLK
HB1_VALID = 128
HB2_VALID = (E - E_HALF) // EBLK

_MESH = plsc.VectorSubcoreMesh(
    core_axis_name="c", subcore_axis_name="s", num_cores=NCORE,
    num_subcores=NSUB)

F32 = jnp.float32
BF16 = jnp.bfloat16


def _lr(x):
    return jnp.where(x >= 0, x, 0.2 * x)


def _dot(a, b):
    return jnp.dot(a, b, preferred_element_type=F32)


GCH = 40
G_NB = 4
G_CHUNKS = EPW_H // GCH
G_OUTER = G_CHUNKS // G_NB
TROWS_SUB = N_TAB // NSUB


def _gather2_body(p1_hbm, p2_hbm, i1_hbm, i2_hbm, g1_hbm, g2_hbm,
                  i1_v, i2_v, bufs, gsems, ssems, tab_sh):
    sid = lax.axis_index("s")
    wid = sid * NCORE + lax.axis_index("c")
    base = wid * EPW_H
    pltpu.sync_copy(i1_hbm.at[pl.ds(base, EPW_H)], i1_v)
    pltpu.sync_copy(i2_hbm.at[pl.ds(base, EPW_H)], i2_v)

    def phase(p_hbm, idxv, g_hbm):
        pltpu.sync_copy(p_hbm.at[pl.ds(sid * TROWS_SUB, TROWS_SUB)],
                        tab_sh.at[pl.ds(sid * TROWS_SUB, TROWS_SUB)])
        plsc.subcore_barrier()

        def issue(c, b):
            pltpu.async_copy(tab_sh.at[idxv.at[pl.ds(c * GCH, GCH)]],
                             bufs[b], gsems[b])

        for b in range(G_NB):
            issue(b, b)

        def outer(jo, carry):
            for b in range(G_NB):
                c = jo * G_NB + b
                goff = base + c * GCH
                pltpu.make_async_copy(
                    tab_sh.at[idxv.at[pl.ds(0, GCH)]], bufs[b],
                    gsems[b]).wait()
                pltpu.async_copy(bufs[b], g_hbm.at[pl.ds(goff, GCH)],
                                 ssems[b])
                pltpu.make_async_copy(
                    bufs[b], g_hbm.at[pl.ds(base, GCH)], ssems[b]).wait()

                @pl.when(jo < G_OUTER - 1)
                def _issue_next():
                    issue(c + G_NB, b)
            return carry

        lax.fori_loop(0, G_OUTER, outer, 0)
        plsc.subcore_barrier()

    phase(p1_hbm, i1_v, g1_hbm)
    phase(p2_hbm, i2_v, g2_hbm)


_gather2 = pl.kernel(
    _gather2_body,
    out_type=[jax.ShapeDtypeStruct((E_HALF, 128), F32),
              jax.ShapeDtypeStruct((E_HALF, 128), F32)],
    mesh=_MESH,
    scratch_types=[
        pltpu.VMEM((EPW_H,), jnp.int32),
        pltpu.VMEM((EPW_H,), jnp.int32),
        [pltpu.VMEM((GCH, 128), F32) for _ in range(G_NB)],
        [pltpu.SemaphoreType.DMA for _ in range(G_NB)],
        [pltpu.SemaphoreType.DMA for _ in range(G_NB)],
        pltpu.VMEM_SHARED((N_TAB, 128), F32),
    ],
)


S_NB = 2
EPW_S = E_HALF // NW
S_CHUNKS = EPW_S // GC
S_OUTER = S_CHUNKS // S_NB


def _scatter_node_body(vals_hbm, idx_hbm, init_hbm, out_hbm,
                       idx_v, vals_v, tab_sh, isem, vsem):
    cid = lax.axis_index("c")
    sid = lax.axis_index("s")
    wid = sid * NCORE + cid
    pltpu.sync_copy(init_hbm.at[pl.ds(cid * N_TAB + sid * NROWS_SUB,
                                      NROWS_SUB)],
                    tab_sh.at[pl.ds(sid * NROWS_SUB, NROWS_SUB)])
    plsc.subcore_barrier()
    base = wid * EPW_S

    def issue(c, b):
        pltpu.async_copy(idx_hbm.at[pl.ds(base + c * GC, GC)],
                         idx_v[b], isem[b])
        pltpu.async_copy(vals_hbm.at[pl.ds(base + c * GC, GC)],
                         vals_v[b], vsem[b])

    for b in range(S_NB):
        issue(b, b)

    def outer(jo, carry):
        for b in range(S_NB):
            c = jo * S_NB + b
            pltpu.make_async_copy(
                idx_hbm.at[pl.ds(base, GC)], idx_v[b], isem[b]).wait()
            pltpu.make_async_copy(
                vals_hbm.at[pl.ds(base, GC)], vals_v[b], vsem[b]).wait()
            pltpu.sync_copy(vals_v[b], tab_sh.at[idx_v[b]], add=True)

            @pl.when(jo < S_OUTER - 1)
            def _issue_next():
                issue(c + S_NB, b)
        return carry

    lax.fori_loop(0, S_OUTER, outer, 0)
    plsc.subcore_barrier()
    pltpu.sync_copy(tab_sh.at[pl.ds(sid * NROWS_SUB, NROWS_SUB)],
                    out_hbm.at[pl.ds(cid * N_TAB + sid * NROWS_SUB,
                                     NROWS_SUB)])


_scatter_node = pl.kernel(
    _scatter_node_body,
    out_type=jax.ShapeDtypeStruct((NCORE * N_TAB, 128), F32),
    mesh=_MESH,
    scratch_types=[
        [pltpu.VMEM((GC,), jnp.int32) for _ in range(S_NB)],
        [pltpu.VMEM((GC, 128), F32) for _ in range(S_NB)],
        pltpu.VMEM_SHARED((N_TAB, 128), F32),
        [pltpu.SemaphoreType.DMA for _ in range(S_NB)],
        [pltpu.SemaphoreType.DMA for _ in range(S_NB)],
    ],
)


def _scatter_cry_body(va_hbm, vb_hbm, vc_hbm, ia_hbm, ib_hbm, ic_hbm,
                      zeros_hbm, out_hbm, ia_v, ib_v, ic_v, va_v, vb_v, vc_v,
                      tab_sh, sema, semb, semc):
    cid = lax.axis_index("c")
    sid = lax.axis_index("s")
    wid = sid * NCORE + cid
    pltpu.sync_copy(zeros_hbm.at[pl.ds(sid * CRY_ROWS_SUB, CRY_ROWS_SUB)],
                    tab_sh.at[pl.ds(sid * CRY_ROWS_SUB, CRY_ROWS_SUB)])
    plsc.subcore_barrier()
    base = wid * XPW

    def step(j, carry):
        off = base + j * XC
        pltpu.sync_copy(ia_hbm.at[pl.ds(off, XC)], ia_v)
        pltpu.sync_copy(ib_hbm.at[pl.ds(off, XC)], ib_v)
        pltpu.sync_copy(ic_hbm.at[pl.ds(off, XC)], ic_v)
        da = pltpu.async_copy(va_hbm.at[pl.ds(off, XC)], va_v, sema)
        db = pltpu.async_copy(vb_hbm.at[pl.ds(off, XC)], vb_v, semb)
        dc = pltpu.async_copy(vc_hbm.at[pl.ds(off, XC)], vc_v, semc)
        da.wait()
        pltpu.sync_copy(va_v, tab_sh.at[ia_v], add=True)
        db.wait()
        pltpu.sync_copy(vb_v, tab_sh.at[ib_v], add=True)
        dc.wait()
        pltpu.sync_copy(vc_v, tab_sh.at[ic_v], add=True)
        return carry

    lax.fori_loop(0, XPW // XC, step, 0)
    plsc.subcore_barrier()
    pltpu.sync_copy(tab_sh.at[pl.ds(sid * CRY_ROWS_SUB, CRY_ROWS_SUB)],
                    out_hbm.at[pl.ds(cid * CRY_TAB + sid * CRY_ROWS_SUB,
                                     CRY_ROWS_SUB)])


_scatter_cry = pl.kernel(
    _scatter_cry_body,
    out_type=jax.ShapeDtypeStruct((NCORE * CRY_TAB, 128), F32),
    mesh=_MESH,
    scratch_types=[
        pltpu.VMEM((XC,), jnp.int32),
        pltpu.VMEM((XC,), jnp.int32),
        pltpu.VMEM((XC,), jnp.int32),
        pltpu.VMEM((XC, 128), F32),
        pltpu.VMEM((XC, 128), F32),
        pltpu.VMEM((XC, 128), F32),
        pltpu.VMEM_SHARED((CRY_TAB, 128), F32),
        pltpu.SemaphoreType.DMA,
        pltpu.SemaphoreType.DMA,
        pltpu.SemaphoreType.DMA,
    ],
)


def _embed_nodes_k(af, nw, nb, w1a, w1b, b1, atom_o, p1_o, p2_o):
    a = _dot(af[...], nw[...]) + nb[...]
    atom_o[...] = a
    p1_o[0:N, :] = _dot(a, w1a[...])
    p1_o[N:N_TAB, :] = jnp.zeros((N_TAB - N, 128), F32)
    p2_o[0:N, :] = _dot(a, w1b[...]) + b1[...]
    p2_o[N:N_TAB, :] = jnp.zeros((N_TAB - N, 128), F32)


def _make_embed_edges_k(nvalid):
    def _embed_edges_k(nf, ew, eb, out):
        blk = pl.program_id(0)
        r = _dot(nf[...], ew[...]) + eb[...]
        out[...] = jnp.where(blk < nvalid, r, 0.0)
    return _embed_edges_k


def _make_edge_mlp_k(nvalid):
    def _edge_mlp_k(nbr, g1, g2, w1c, w2, b2, w3, b3, out):
        blk = pl.program_id(0)
        h = _lr(g1[...] + g2[...] + _dot(nbr[...].astype(BF16), w1c[...]))
        h = _lr(_dot(h.astype(BF16), w2[...]) + b2[...])
        ek = _dot(h.astype(BF16), w3[...]) + b3[...]
        out[...] = jnp.where(blk < nvalid, nbr[...] + ek, 0.0)
    return _edge_mlp_k


def _sum_partials_k(pa, out):
    out[...] = pa[0:N, :] + pa[N_TAB:N_TAB + N, :]


def _sum_partials(pa):
    return pl.pallas_call(
        _sum_partials_k,
        out_shape=jax.ShapeDtypeStruct((N, 128), F32),
    )(pa)


def _node_update_body(last, atom, s_new_ref, s_old_ref, inv_nn,
                      pva, pvb, pvb1, pv2, pv2b, pv3, pv3b, bng, bnb,
                      w1a, w1b, b1, *outs):
    s_new = s_new_ref[...]
    s_old = s_old_ref[...]
    rho = (s_new - s_old) * inv_nn[...]
    a = atom[...]
    vi = _lr(_dot(a, pva[...]) + _dot(rho, pvb[...]) + pvb1[...])
    vi = _lr(_dot(vi, pv2[...]) + pv2b[...])
    vi = _dot(vi, pv3[...]) + pv3b[...]
    m = jnp.mean(vi, axis=0, keepdims=True)
    v = jnp.mean((vi - m) ** 2, axis=0, keepdims=True)
    vi = bng[...] * (vi - m) / jnp.sqrt(v + 1e-5) + bnb[...]
    a_new = a + vi
    if last:
        atom_o, ssum_o, anfa_o, anfb_o = outs
        anfa_o[0:N, :] = a_new
        anfa_o[N:X_ROWS, :] = jnp.zeros((X_ROWS - N, 128), F32)
        anfb_o[0:N, :] = s_new * inv_nn[...]
        anfb_o[N:X_ROWS, :] = jnp.zeros((X_ROWS - N, 128), F32)
    else:
        atom_o, ssum_o, p1_o, p2_o = outs
        p1_o[0:N, :] = _dot(a_new, w1a[...])
        p1_o[N:N_TAB, :] = jnp.zeros((N_TAB - N, 128), F32)
        p2_o[0:N, :] = _dot(a_new, w1b[...]) + b1[...]
        p2_o[N:N_TAB, :] = jnp.zeros((N_TAB - N, 128), F32)
    atom_o[...] = a_new
    ssum_o[...] = s_new


def _readout_k(tabp, unrel, rel, cell, t_mats, e_mats,
               puw1a, puw1b, pub1, puw2, pub2, zcwz, zcwu, zcwr, zcb, zcg,
               zcbeta, c2fw, c2fb, fc1w, fc1b, fow1, fob1, fow2, fob2, fow3,
               fob3, olw1, olb1, olw2, olb2, out_o, z_o):
    tab_a = tabp[0:NCRY, :] + tabp[CRY_TAB:CRY_TAB + NCRY, :]
    tab_b = (tabp[NCRY:2 * NCRY, :]
             + tabp[CRY_TAB + NCRY:CRY_TAB + 2 * NCRY, :])
    cnt = (tabp[2 * NCRY:3 * NCRY, :]
           + tabp[CRY_TAB + 2 * NCRY:CRY_TAB + 3 * NCRY, :])[:, 0:1]
    ga = tab_a / cnt
    gb = tab_b / cnt
    z = jnp.tanh(_dot(_lr(_dot(ga, puw1a[...]) + _dot(gb, puw1b[...])
                          + pub1[...]), puw2[...])
                 + pub2[...])
    z_o[...] = z
    zz = _lr(_dot(z, zcwz[...]) + _dot(unrel[...], zcwu[...])
             + _dot(rel[...], zcwr[...]) + zcb[...])
    m = jnp.mean(zz, axis=0, keepdims=True)
    v = jnp.mean((zz - m) ** 2, axis=0, keepdims=True)
    zz = zcg[...] * (zz - m) / jnp.sqrt(v + 1e-5) + zcbeta[...]
    crys = _lr(_dot(zz, c2fw[...]) + c2fb[...])
    crys = _lr(_dot(crys, fc1w[...]) + fc1b[...])
    o = _lr(_dot(crys, fow1[...]) + fob1[...])
    o = _lr(_dot(o, fow2[...]) + fob2[...])
    o = _lr(_dot(o, fow3[...]) + fob3[...])
    c = cell[...]
    res = ((_dot(o, t_mats[0]) * _dot(c, e_mats[0]))
           + (_dot(o, t_mats[1]) * _dot(c, e_mats[1]))
           + (_dot(o, t_mats[2]) * _dot(c, e_mats[2])))
    o = _lr(_dot(res, olw1[...]) + olb1[...])
    out_o[...] = _dot(o, olw2[...]) + olb2[...]


_T_MATS = np.zeros((3, 192, 192), np.float32)
_E_MATS = np.zeros((3, 9, 192), np.float32)
for _k in range(3):
    for _i in range(64):
        for _j in range(3):
            _T_MATS[_k, 3 * _i + _k, 3 * _i + _j] = 1.0
            _E_MATS[_k, 3 * _k + _j, 3 * _i + _j] = 1.0


def _full(shape):
    return pl.BlockSpec(shape, lambda i: (0, 0))


def _embed_nodes(af, nw, nb, w1a, w1b, b1):
    return pl.pallas_call(
        _embed_nodes_k,
        out_shape=[jax.ShapeDtypeStruct((N, 128), F32),
                   jax.ShapeDtypeStruct((N_TAB, 128), F32),
                   jax.ShapeDtypeStruct((N_TAB, 128), F32)],
    )(af, nw, nb, w1a, w1b, b1)


def _embed_edges(nf_half, nvalid):
    def run(ew, eb):
        return pl.pallas_call(
            _make_embed_edges_k(nvalid),
            grid=(N_EBLK_H,),
            in_specs=[
                pl.BlockSpec((EBLK, 16), lambda i: (i, 0)),
                _full((16, 128)),
                _full((1, 128)),
            ],
            out_specs=pl.BlockSpec((EBLK, 128), lambda i: (i, 0)),
            out_shape=jax.ShapeDtypeStruct((E_HALF, 128), F32),
        )(nf_half, ew, eb)
    return run


def _edge_mlp(nbr, g1, g2, w1c, w2, b2, w3, b3, nvalid):
    return pl.pallas_call(
        _make_edge_mlp_k(nvalid),
        grid=(N_EBLK_H,),
        in_specs=[
            pl.BlockSpec((EBLK, 128), lambda i: (i, 0)),
            pl.BlockSpec((EBLK, 128), lambda i: (i, 0)),
            pl.BlockSpec((EBLK, 128), lambda i: (i, 0)),
            _full((128, 128)),
            _full((128, 128)),
            _full((1, 128)),
            _full((128, 128)),
            _full((1, 128)),
        ],
        out_specs=pl.BlockSpec((EBLK, 128), lambda i: (i, 0)),
        out_shape=jax.ShapeDtypeStruct((E_HALF, 128), F32),
    )(nbr, g1, g2, w1c, w2, b2, w3, b3)


def _node_update(last, atom, s_new, s_old, inv_nn, pva, pvb, pvb1, pv2,
                 pv2b, pv3, pv3b, bng, bnb, w1a, w1b, b1):
    if last:
        out_shape = [jax.ShapeDtypeStruct((N, 128), F32),
                     jax.ShapeDtypeStruct((N, 128), F32),
                     jax.ShapeDtypeStruct((X_ROWS, 128), F32),
                     jax.ShapeDtypeStruct((X_ROWS, 128), F32)]
    else:
        out_shape = [jax.ShapeDtypeStruct((N, 128), F32),
                     jax.ShapeDtypeStruct((N, 128), F32),
                     jax.ShapeDtypeStruct((N_TAB, 128), F32),
                     jax.ShapeDtypeStruct((N_TAB, 128), F32)]
    return pl.pallas_call(
        functools.partial(_node_update_body, last),
        out_shape=out_shape,
    )(atom, s_new, s_old, inv_nn, pva, pvb, pvb1, pv2, pv2b, pv3, pv3b,
      bng, bnb, w1a, w1b, b1)


def _readout(tabp, unrel, rel, cell, t_mats, e_mats, p):
    zw = p["zc_W"]
    return pl.pallas_call(
        _readout_k,
        out_shape=[jax.ShapeDtypeStruct((NCRY, 9), F32),
                   jax.ShapeDtypeStruct((NCRY, 256), F32)],
    )(tabp, unrel, rel, cell, t_mats, e_mats,
      p["pu_W1"][:128], p["pu_W1"][128:], p["pu_b1"].reshape(1, -1),
      p["pu_W2"], p["pu_b2"].reshape(1, -1),
      zw[:256], zw[256:320], zw[320:], p["zc_b"].reshape(1, -1),
      p["zc_g"].reshape(1, -1), p["zc_beta"].reshape(1, -1),
      p["c2f_W"], p["c2f_b"].reshape(1, -1),
      p["fc1_W"], p["fc1_b"].reshape(1, -1),
      p["fo_W1"], p["fo_b1"].reshape(1, -1),
      p["fo_W2"], p["fo_b2"].reshape(1, -1),
      p["fo_W3"], p["fo_b3"].reshape(1, -1),
      p["ol_W1"], p["ol_b1"].reshape(1, -1),
      p["ol_W2"], p["ol_b2"].reshape(1, -1))


def kernel(atom_fea, nbr_fea, nbr_fea_idx1, nbr_fea_idx2, num_nbrs,
           crystal_atom_idx, unrelaxed_feature, relaxed_feature, cell, delta,
           params):
    p = params
    idx1 = jnp.concatenate(
        [nbr_fea_idx1.astype(jnp.int32),
         jnp.zeros((E_PAD - E,), jnp.int32)])
    idx2 = jnp.concatenate(
        [nbr_fea_idx2.astype(jnp.int32),
         jnp.zeros((E_PAD - E,), jnp.int32)])
    nf_pad = jnp.pad(nbr_fea, ((0, E_PAD - E), (0, 0)))
    i1h = (idx1[:E_HALF], idx1[E_HALF:])
    i2h = (idx2[:E_HALF], idx2[E_HALF:])
    inv_nn = (1.0 / num_nbrs).reshape(N, 1)
    zeros_n = jnp.zeros((NCORE * N_TAB, 128), F32)
    zeros_c = jnp.zeros((CRY_TAB, 128), F32)
    t_mats = jnp.asarray(_T_MATS)
    e_mats = jnp.asarray(_E_MATS)

    w1 = [p["pe_W1"][i] for i in range(NCONV)]
    b1 = [p["pe_b1"][i].reshape(1, -1) for i in range(NCONV)]

    atom, p1, p2 = _embed_nodes(
        atom_fea, p["node_W"], p["node_b"].reshape(1, -1),
        w1[0][:128], w1[0][128:256], b1[0])
    ew, eb = p["edge_W"], p["edge_b"].reshape(1, -1)
    nbr_a = _embed_edges(nf_pad[:E_HALF], HB1_VALID)(ew, eb)
    nbr_b = _embed_edges(nf_pad[E_HALF:], HB2_VALID)(ew, eb)
    s0a = _scatter_node(nbr_a, i1h[0], zeros_n)
    s0b = _scatter_node(nbr_b, i1h[1], s0a)
    s_old = _sum_partials(s0b)

    anf = None
    for i in range(NCONV):
        last = i == NCONV - 1
        ek_w = (w1[i][256:].astype(BF16), p["pe_W2"][i].astype(BF16),
                p["pe_b2"][i].reshape(1, -1),
                p["pe_W3"][i].astype(BF16), p["pe_b3"][i].reshape(1, -1))
        g1a, g2a = _gather2(p1, p2, i1h[0], i2h[0])
        g1b, g2b = _gather2(p1, p2, i1h[1], i2h[1])
        nbr_a = _edge_mlp(nbr_a, g1a, g2a, *ek_w, HB1_VALID)
        sna = _scatter_node(nbr_a, i1h[0], zeros_n)
        nbr_b = _edge_mlp(nbr_b, g1b, g2b, *ek_w, HB2_VALID)
        snb = _scatter_node(nbr_b, i1h[1], sna)
        pv = p["pv_W1"][i]
        if last:
            nw1a = nw1b = jnp.zeros((128, 128), F32)
            nb1 = jnp.zeros((1, 128), F32)
        else:
            nw1a, nw1b, nb1 = w1[i + 1][:128], w1[i + 1][128:256], b1[i + 1]
        s_new = _sum_partials(snb)
        outs = _node_update(
            last, atom, s_new, s_old, inv_nn,
            pv[:128], pv[128:], p["pv_b1"][i].reshape(1, -1),
            p["pv_W2"][i], p["pv_b2"][i].reshape(1, -1),
            p["pv_W3"][i], p["pv_b3"][i].reshape(1, -1),
            p["bn_g"][i].reshape(1, -1), p["bn_b"][i].reshape(1, -1),
            nw1a, nw1b, nb1)
        if last:
            atom, ssum, anf_a, anf_b = outs
        else:
            atom, ssum, p1, p2 = outs
        s_old = ssum

    ones_col = jnp.zeros((X_ROWS, 128), F32).at[:N, 0].set(1.0)
    cry_idx = jnp.concatenate(
        [crystal_atom_idx.astype(jnp.int32),
         jnp.zeros((X_ROWS - N,), jnp.int32)])
    tabp = _scatter_cry(anf_a, anf_b, ones_col, cry_idx, cry_idx + NCRY,
                        cry_idx + 2 * NCRY, zeros_c)

    out, z = _readout(tabp, unrelaxed_feature, relaxed_feature, cell,
                      t_mats, e_mats, p)
    return (out, z)


# speedup vs baseline: 1.0041x; 1.0041x over previous
import functools

import jax
import jax.numpy as jnp
import numpy as np
from jax import lax
from jax.experimental import pallas as pl
from jax.experimental.pallas import tpu as pltpu
from jax.experimental.pallas import tpu_sc as plsc

N = 10000
E = 320000
NCRY = 512
NCONV = 3

NCORE = 2
NSUB = 16
NW = NCORE * NSUB

E_PAD = 327680
EPW = E_PAD // NW
GC = 128

N_TAB = 10240
NROWS_SUB = N_TAB // NSUB

CRY_TAB = 3 * NCRY
CRY_ROWS_SUB = CRY_TAB // NSUB
X_ROWS = 10240
XPW = X_ROWS // NW
XC = 64

EBLK = 1280
E_HALF = E_PAD // 2
EPW_H = E_HALF // NW
N_EBLK_H = E_HALF // EBLK
HB1_VALID = 128
HB2_VALID = (E - E_HALF) // EBLK

_MESH = plsc.VectorSubcoreMesh(
    core_axis_name="c", subcore_axis_name="s", num_cores=NCORE,
    num_subcores=NSUB)

F32 = jnp.float32
BF16 = jnp.bfloat16


def _lr(x):
    return jnp.where(x >= 0, x, 0.2 * x)


def _dot(a, b):
    return jnp.dot(a, b, preferred_element_type=F32)


GCH = 40
G_NB = 4
G_CHUNKS = EPW_H // GCH
G_OUTER = G_CHUNKS // G_NB
TROWS_SUB = N_TAB // NSUB


def _gather2_body(p1_hbm, p2_hbm, i1_hbm, i2_hbm, g1_hbm, g2_hbm,
                  i1_v, i2_v, bufs, gsems, ssems, tab_sh):
    sid = lax.axis_index("s")
    wid = sid * NCORE + lax.axis_index("c")
    base = wid * EPW_H
    pltpu.sync_copy(i1_hbm.at[pl.ds(base, EPW_H)], i1_v)
    pltpu.sync_copy(i2_hbm.at[pl.ds(base, EPW_H)], i2_v)

    def phase(p_hbm, idxv, g_hbm):
        pltpu.sync_copy(p_hbm.at[pl.ds(sid * TROWS_SUB, TROWS_SUB)],
                        tab_sh.at[pl.ds(sid * TROWS_SUB, TROWS_SUB)])
        plsc.subcore_barrier()

        def issue(c, b):
            pltpu.async_copy(tab_sh.at[idxv.at[pl.ds(c * GCH, GCH)]],
                             bufs[b], gsems[b])

        for b in range(G_NB):
            issue(b, b)

        def outer(jo, carry):
            for b in range(G_NB):
                c = jo * G_NB + b
                goff = base + c * GCH
                pltpu.make_async_copy(
                    tab_sh.at[idxv.at[pl.ds(0, GCH)]], bufs[b],
                    gsems[b]).wait()
                pltpu.async_copy(bufs[b], g_hbm.at[pl.ds(goff, GCH)],
                                 ssems[b])
                pltpu.make_async_copy(
                    bufs[b], g_hbm.at[pl.ds(base, GCH)], ssems[b]).wait()

                @pl.when(jo < G_OUTER - 1)
                def _issue_next():
                    issue(c + G_NB, b)
            return carry

        lax.fori_loop(0, G_OUTER, outer, 0)
        plsc.subcore_barrier()

    phase(p1_hbm, i1_v, g1_hbm)
    phase(p2_hbm, i2_v, g2_hbm)


_gather2 = pl.kernel(
    _gather2_body,
    out_type=[jax.ShapeDtypeStruct((E_HALF, 128), F32),
              jax.ShapeDtypeStruct((E_HALF, 128), F32)],
    mesh=_MESH,
    scratch_types=[
        pltpu.VMEM((EPW_H,), jnp.int32),
        pltpu.VMEM((EPW_H,), jnp.int32),
        [pltpu.VMEM((GCH, 128), F32) for _ in range(G_NB)],
        [pltpu.SemaphoreType.DMA for _ in range(G_NB)],
        [pltpu.SemaphoreType.DMA for _ in range(G_NB)],
        pltpu.VMEM_SHARED((N_TAB, 128), F32),
    ],
)


S_NB = 2
EPW_S = E_HALF // NW
S_CHUNKS = EPW_S // GC
S_OUTER = S_CHUNKS // S_NB


def _scatter_node_body(vals_hbm, idx_hbm, init_hbm, out_hbm,
                       idx_v, vals_v, tab_sh, isem, vsem):
    cid = lax.axis_index("c")
    sid = lax.axis_index("s")
    wid = sid * NCORE + cid
    pltpu.sync_copy(init_hbm.at[pl.ds(cid * N_TAB + sid * NROWS_SUB,
                                      NROWS_SUB)],
                    tab_sh.at[pl.ds(sid * NROWS_SUB, NROWS_SUB)])
    plsc.subcore_barrier()
    base = wid * EPW_S

    def issue(c, b):
        pltpu.async_copy(idx_hbm.at[pl.ds(base + c * GC, GC)],
                         idx_v[b], isem[b])
        pltpu.async_copy(vals_hbm.at[pl.ds(base + c * GC, GC)],
                         vals_v[b], vsem[b])

    for b in range(S_NB):
        issue(b, b)

    def outer(jo, carry):
        for b in range(S_NB):
            c = jo * S_NB + b
            pltpu.make_async_copy(
                idx_hbm.at[pl.ds(base, GC)], idx_v[b], isem[b]).wait()
            pltpu.make_async_copy(
                vals_hbm.at[pl.ds(base, GC)], vals_v[b], vsem[b]).wait()
            pltpu.sync_copy(vals_v[b], tab_sh.at[idx_v[b]], add=True)

            @pl.when(jo < S_OUTER - 1)
            def _issue_next():
                issue(c + S_NB, b)
        return carry

    lax.fori_loop(0, S_OUTER, outer, 0)
    plsc.subcore_barrier()
    pltpu.sync_copy(tab_sh.at[pl.ds(sid * NROWS_SUB, NROWS_SUB)],
                    out_hbm.at[pl.ds(cid * N_TAB + sid * NROWS_SUB,
                                     NROWS_SUB)])


_scatter_node = pl.kernel(
    _scatter_node_body,
    out_type=jax.ShapeDtypeStruct((NCORE * N_TAB, 128), F32),
    mesh=_MESH,
    scratch_types=[
        [pltpu.VMEM((GC,), jnp.int32) for _ in range(S_NB)],
        [pltpu.VMEM((GC, 128), F32) for _ in range(S_NB)],
        pltpu.VMEM_SHARED((N_TAB, 128), F32),
        [pltpu.SemaphoreType.DMA for _ in range(S_NB)],
        [pltpu.SemaphoreType.DMA for _ in range(S_NB)],
    ],
)


def _scatter_cry_body(va_hbm, vb_hbm, vc_hbm, ia_hbm, ib_hbm, ic_hbm,
                      zeros_hbm, out_hbm, ia_v, ib_v, ic_v, va_v, vb_v, vc_v,
                      tab_sh, sema, semb, semc):
    cid = lax.axis_index("c")
    sid = lax.axis_index("s")
    wid = sid * NCORE + cid
    pltpu.sync_copy(zeros_hbm.at[pl.ds(sid * CRY_ROWS_SUB, CRY_ROWS_SUB)],
                    tab_sh.at[pl.ds(sid * CRY_ROWS_SUB, CRY_ROWS_SUB)])
    plsc.subcore_barrier()
    base = wid * XPW

    def step(j, carry):
        off = base + j * XC
        pltpu.sync_copy(ia_hbm.at[pl.ds(off, XC)], ia_v)
        pltpu.sync_copy(ib_hbm.at[pl.ds(off, XC)], ib_v)
        pltpu.sync_copy(ic_hbm.at[pl.ds(off, XC)], ic_v)
        da = pltpu.async_copy(va_hbm.at[pl.ds(off, XC)], va_v, sema)
        db = pltpu.async_copy(vb_hbm.at[pl.ds(off, XC)], vb_v, semb)
        dc = pltpu.async_copy(vc_hbm.at[pl.ds(off, XC)], vc_v, semc)
        da.wait()
        pltpu.sync_copy(va_v, tab_sh.at[ia_v], add=True)
        db.wait()
        pltpu.sync_copy(vb_v, tab_sh.at[ib_v], add=True)
        dc.wait()
        pltpu.sync_copy(vc_v, tab_sh.at[ic_v], add=True)
        return carry

    lax.fori_loop(0, XPW // XC, step, 0)
    plsc.subcore_barrier()
    pltpu.sync_copy(tab_sh.at[pl.ds(sid * CRY_ROWS_SUB, CRY_ROWS_SUB)],
                    out_hbm.at[pl.ds(cid * CRY_TAB + sid * CRY_ROWS_SUB,
                                     CRY_ROWS_SUB)])


_scatter_cry = pl.kernel(
    _scatter_cry_body,
    out_type=jax.ShapeDtypeStruct((NCORE * CRY_TAB, 128), F32),
    mesh=_MESH,
    scratch_types=[
        pltpu.VMEM((XC,), jnp.int32),
        pltpu.VMEM((XC,), jnp.int32),
        pltpu.VMEM((XC,), jnp.int32),
        pltpu.VMEM((XC, 128), F32),
        pltpu.VMEM((XC, 128), F32),
        pltpu.VMEM((XC, 128), F32),
        pltpu.VMEM_SHARED((CRY_TAB, 128), F32),
        pltpu.SemaphoreType.DMA,
        pltpu.SemaphoreType.DMA,
        pltpu.SemaphoreType.DMA,
    ],
)


def _embed_nodes_k(af, nw, nb, w1a, w1b, b1, atom_o, p1_o, p2_o):
    a = _dot(af[...], nw[...]) + nb[...]
    atom_o[...] = a
    p1_o[0:N, :] = _dot(a, w1a[...])
    p1_o[N:N_TAB, :] = jnp.zeros((N_TAB - N, 128), F32)
    p2_o[0:N, :] = _dot(a, w1b[...]) + b1[...]
    p2_o[N:N_TAB, :] = jnp.zeros((N_TAB - N, 128), F32)


def _make_embed_edges_k(nvalid):
    def _embed_edges_k(nf, ew, eb, out):
        blk = pl.program_id(0)
        r = _dot(nf[...], ew[...]) + eb[...]
        out[...] = jnp.where(blk < nvalid, r, 0.0)
    return _embed_edges_k


def _make_edge_mlp_k(nvalid):
    def _edge_mlp_k(nbr, g1, g2, w1c, w2, b2, w3, b3, out):
        blk = pl.program_id(0)
        h = _lr(g1[...] + g2[...] + _dot(nbr[...], w1c[...]))
        h = _lr(_dot(h, w2[...]) + b2[...])
        ek = _dot(h, w3[...]) + b3[...]
        out[...] = jnp.where(blk < nvalid, nbr[...] + ek, 0.0)
    return _edge_mlp_k


def _sum_partials_k(pa, out):
    out[...] = pa[0:N, :] + pa[N_TAB:N_TAB + N, :]


def _sum_partials(pa):
    return pl.pallas_call(
        _sum_partials_k,
        out_shape=jax.ShapeDtypeStruct((N, 128), F32),
    )(pa)


def _node_update_body(last, atom, s_new_ref, s_old_ref, inv_nn,
                      pva, pvb, pvb1, pv2, pv2b, pv3, pv3b, bng, bnb,
                      w1a, w1b, b1, *outs):
    s_new = s_new_ref[...]
    s_old = s_old_ref[...]
    rho = (s_new - s_old) * inv_nn[...]
    a = atom[...]
    vi = _lr(_dot(a, pva[...]) + _dot(rho, pvb[...]) + pvb1[...])
    vi = _lr(_dot(vi, pv2[...]) + pv2b[...])
    vi = _dot(vi, pv3[...]) + pv3b[...]
    m = jnp.mean(vi, axis=0, keepdims=True)
    v = jnp.mean((vi - m) ** 2, axis=0, keepdims=True)
    vi = bng[...] * (vi - m) / jnp.sqrt(v + 1e-5) + bnb[...]
    a_new = a + vi
    if last:
        atom_o, ssum_o, anfa_o, anfb_o = outs
        anfa_o[0:N, :] = a_new
        anfa_o[N:X_ROWS, :] = jnp.zeros((X_ROWS - N, 128), F32)
        anfb_o[0:N, :] = s_new * inv_nn[...]
        anfb_o[N:X_ROWS, :] = jnp.zeros((X_ROWS - N, 128), F32)
    else:
        atom_o, ssum_o, p1_o, p2_o = outs
        p1_o[0:N, :] = _dot(a_new, w1a[...])
        p1_o[N:N_TAB, :] = jnp.zeros((N_TAB - N, 128), F32)
        p2_o[0:N, :] = _dot(a_new, w1b[...]) + b1[...]
        p2_o[N:N_TAB, :] = jnp.zeros((N_TAB - N, 128), F32)
    atom_o[...] = a_new
    ssum_o[...] = s_new


def _readout_k(tabp, unrel, rel, cell, t_mats, e_mats,
               puw1a, puw1b, pub1, puw2, pub2, zcwz, zcwu, zcwr, zcb, zcg,
               zcbeta, c2fw, c2fb, fc1w, fc1b, fow1, fob1, fow2, fob2, fow3,
               fob3, olw1, olb1, olw2, olb2, out_o, z_o):
    tab_a = tabp[0:NCRY, :] + tabp[CRY_TAB:CRY_TAB + NCRY, :]
    tab_b = (tabp[NCRY:2 * NCRY, :]
             + tabp[CRY_TAB + NCRY:CRY_TAB + 2 * NCRY, :])
    cnt = (tabp[2 * NCRY:3 * NCRY, :]
           + tabp[CRY_TAB + 2 * NCRY:CRY_TAB + 3 * NCRY, :])[:, 0:1]
    ga = tab_a / cnt
    gb = tab_b / cnt
    z = jnp.tanh(_dot(_lr(_dot(ga, puw1a[...]) + _dot(gb, puw1b[...])
                          + pub1[...]), puw2[...])
                 + pub2[...])
    z_o[...] = z
    zz = _lr(_dot(z, zcwz[...]) + _dot(unrel[...], zcwu[...])
             + _dot(rel[...], zcwr[...]) + zcb[...])
    m = jnp.mean(zz, axis=0, keepdims=True)
    v = jnp.mean((zz - m) ** 2, axis=0, keepdims=True)
    zz = zcg[...] * (zz - m) / jnp.sqrt(v + 1e-5) + zcbeta[...]
    crys = _lr(_dot(zz, c2fw[...]) + c2fb[...])
    crys = _lr(_dot(crys, fc1w[...]) + fc1b[...])
    o = _lr(_dot(crys, fow1[...]) + fob1[...])
    o = _lr(_dot(o, fow2[...]) + fob2[...])
    o = _lr(_dot(o, fow3[...]) + fob3[...])
    c = cell[...]
    res = ((_dot(o, t_mats[0]) * _dot(c, e_mats[0]))
           + (_dot(o, t_mats[1]) * _dot(c, e_mats[1]))
           + (_dot(o, t_mats[2]) * _dot(c, e_mats[2])))
    o = _lr(_dot(res, olw1[...]) + olb1[...])
    out_o[...] = _dot(o, olw2[...]) + olb2[...]


_T_MATS = np.zeros((3, 192, 192), np.float32)
_E_MATS = np.zeros((3, 9, 192), np.float32)
for _k in range(3):
    for _i in range(64):
        for _j in range(3):
            _T_MATS[_k, 3 * _i + _k, 3 * _i + _j] = 1.0
            _E_MATS[_k, 3 * _k + _j, 3 * _i + _j] = 1.0


def _full(shape):
    return pl.BlockSpec(shape, lambda i: (0, 0))


def _embed_nodes(af, nw, nb, w1a, w1b, b1):
    return pl.pallas_call(
        _embed_nodes_k,
        out_shape=[jax.ShapeDtypeStruct((N, 128), F32),
                   jax.ShapeDtypeStruct((N_TAB, 128), F32),
                   jax.ShapeDtypeStruct((N_TAB, 128), F32)],
    )(af, nw, nb, w1a, w1b, b1)


def _embed_edges(nf_half, nvalid):
    def run(ew, eb):
        return pl.pallas_call(
            _make_embed_edges_k(nvalid),
            grid=(N_EBLK_H,),
            in_specs=[
                pl.BlockSpec((EBLK, 16), lambda i: (i, 0)),
                _full((16, 128)),
                _full((1, 128)),
            ],
            out_specs=pl.BlockSpec((EBLK, 128), lambda i: (i, 0)),
            out_shape=jax.ShapeDtypeStruct((E_HALF, 128), F32),
        )(nf_half, ew, eb)
    return run


def _edge_mlp(nbr, g1, g2, w1c, w2, b2, w3, b3, nvalid):
    return pl.pallas_call(
        _make_edge_mlp_k(nvalid),
        grid=(N_EBLK_H,),
        in_specs=[
            pl.BlockSpec((EBLK, 128), lambda i: (i, 0)),
            pl.BlockSpec((EBLK, 128), lambda i: (i, 0)),
            pl.BlockSpec((EBLK, 128), lambda i: (i, 0)),
            _full((128, 128)),
            _full((128, 128)),
            _full((1, 128)),
            _full((128, 128)),
            _full((1, 128)),
        ],
        out_specs=pl.BlockSpec((EBLK, 128), lambda i: (i, 0)),
        out_shape=jax.ShapeDtypeStruct((E_HALF, 128), F32),
    )(nbr, g1, g2, w1c, w2, b2, w3, b3)


def _node_update(last, atom, s_new, s_old, inv_nn, pva, pvb, pvb1, pv2,
                 pv2b, pv3, pv3b, bng, bnb, w1a, w1b, b1):
    if last:
        out_shape = [jax.ShapeDtypeStruct((N, 128), F32),
                     jax.ShapeDtypeStruct((N, 128), F32),
                     jax.ShapeDtypeStruct((X_ROWS, 128), F32),
                     jax.ShapeDtypeStruct((X_ROWS, 128), F32)]
    else:
        out_shape = [jax.ShapeDtypeStruct((N, 128), F32),
                     jax.ShapeDtypeStruct((N, 128), F32),
                     jax.ShapeDtypeStruct((N_TAB, 128), F32),
                     jax.ShapeDtypeStruct((N_TAB, 128), F32)]
    return pl.pallas_call(
        functools.partial(_node_update_body, last),
        out_shape=out_shape,
    )(atom, s_new, s_old, inv_nn, pva, pvb, pvb1, pv2, pv2b, pv3, pv3b,
      bng, bnb, w1a, w1b, b1)


def _readout(tabp, unrel, rel, cell, t_mats, e_mats, p):
    zw = p["zc_W"]
    return pl.pallas_call(
        _readout_k,
        out_shape=[jax.ShapeDtypeStruct((NCRY, 9), F32),
                   jax.ShapeDtypeStruct((NCRY, 256), F32)],
    )(tabp, unrel, rel, cell, t_mats, e_mats,
      p["pu_W1"][:128], p["pu_W1"][128:], p["pu_b1"].reshape(1, -1),
      p["pu_W2"], p["pu_b2"].reshape(1, -1),
      zw[:256], zw[256:320], zw[320:], p["zc_b"].reshape(1, -1),
      p["zc_g"].reshape(1, -1), p["zc_beta"].reshape(1, -1),
      p["c2f_W"], p["c2f_b"].reshape(1, -1),
      p["fc1_W"], p["fc1_b"].reshape(1, -1),
      p["fo_W1"], p["fo_b1"].reshape(1, -1),
      p["fo_W2"], p["fo_b2"].reshape(1, -1),
      p["fo_W3"], p["fo_b3"].reshape(1, -1),
      p["ol_W1"], p["ol_b1"].reshape(1, -1),
      p["ol_W2"], p["ol_b2"].reshape(1, -1))


def kernel(atom_fea, nbr_fea, nbr_fea_idx1, nbr_fea_idx2, num_nbrs,
           crystal_atom_idx, unrelaxed_feature, relaxed_feature, cell, delta,
           params):
    p = params
    idx1 = jnp.concatenate(
        [nbr_fea_idx1.astype(jnp.int32),
         jnp.zeros((E_PAD - E,), jnp.int32)])
    idx2 = jnp.concatenate(
        [nbr_fea_idx2.astype(jnp.int32),
         jnp.zeros((E_PAD - E,), jnp.int32)])
    nf_pad = jnp.pad(nbr_fea, ((0, E_PAD - E), (0, 0)))
    i1h = (idx1[:E_HALF], idx1[E_HALF:])
    i2h = (idx2[:E_HALF], idx2[E_HALF:])
    inv_nn = (1.0 / num_nbrs).reshape(N, 1)
    zeros_n = jnp.zeros((NCORE * N_TAB, 128), F32)
    zeros_c = jnp.zeros((CRY_TAB, 128), F32)
    t_mats = jnp.asarray(_T_MATS)
    e_mats = jnp.asarray(_E_MATS)

    w1 = [p["pe_W1"][i] for i in range(NCONV)]
    b1 = [p["pe_b1"][i].reshape(1, -1) for i in range(NCONV)]

    atom, p1, p2 = _embed_nodes(
        atom_fea, p["node_W"], p["node_b"].reshape(1, -1),
        w1[0][:128], w1[0][128:256], b1[0])
    ew, eb = p["edge_W"], p["edge_b"].reshape(1, -1)
    nbr_a = _embed_edges(nf_pad[:E_HALF], HB1_VALID)(ew, eb)
    nbr_b = _embed_edges(nf_pad[E_HALF:], HB2_VALID)(ew, eb)
    s0a = _scatter_node(nbr_a, i1h[0], zeros_n)
    s0b = _scatter_node(nbr_b, i1h[1], s0a)
    s_old = _sum_partials(s0b)

    anf = None
    for i in range(NCONV):
        last = i == NCONV - 1
        ek_w = (w1[i][256:], p["pe_W2"][i], p["pe_b2"][i].reshape(1, -1),
                p["pe_W3"][i], p["pe_b3"][i].reshape(1, -1))
        g1a, g2a = _gather2(p1, p2, i1h[0], i2h[0])
        g1b, g2b = _gather2(p1, p2, i1h[1], i2h[1])
        nbr_a = _edge_mlp(nbr_a, g1a, g2a, *ek_w, HB1_VALID)
        sna = _scatter_node(nbr_a, i1h[0], zeros_n)
        nbr_b = _edge_mlp(nbr_b, g1b, g2b, *ek_w, HB2_VALID)
        snb = _scatter_node(nbr_b, i1h[1], sna)
        pv = p["pv_W1"][i]
        if last:
            nw1a = nw1b = jnp.zeros((128, 128), F32)
            nb1 = jnp.zeros((1, 128), F32)
        else:
            nw1a, nw1b, nb1 = w1[i + 1][:128], w1[i + 1][128:256], b1[i + 1]
        s_new = _sum_partials(snb)
        outs = _node_update(
            last, atom, s_new, s_old, inv_nn,
            pv[:128], pv[128:], p["pv_b1"][i].reshape(1, -1),
            p["pv_W2"][i], p["pv_b2"][i].reshape(1, -1),
            p["pv_W3"][i], p["pv_b3"][i].reshape(1, -1),
            p["bn_g"][i].reshape(1, -1), p["bn_b"][i].reshape(1, -1),
            nw1a, nw1b, nb1)
        if last:
            atom, ssum, anf_a, anf_b = outs
        else:
            atom, ssum, p1, p2 = outs
        s_old = ssum

    ones_col = jnp.zeros((X_ROWS, 128), F32).at[:N, 0].set(1.0)
    cry_idx = jnp.concatenate(
        [crystal_atom_idx.astype(jnp.int32),
         jnp.zeros((X_ROWS - N,), jnp.int32)])
    tabp = _scatter_cry(anf_a, anf_b, ones_col, cry_idx, cry_idx + NCRY,
                        cry_idx + 2 * NCRY, zeros_c)

    out, z = _readout(tabp, unrelaxed_feature, relaxed_feature, cell,
                      t_mats, e_mats, p)
    return (out, z)
